# baseline (speedup 1.0000x reference)
import functools

import jax
import jax.numpy as jnp
from jax import lax
from jax.experimental import pallas as pl
from jax.experimental.pallas import tpu as pltpu
from jax.experimental.pallas import tpu_sc as plsc

B = 1024
EC = 32
N = 200
V = 100000
RB = 8
CB = 8192
GI = B // RB
GJ = (V + CB - 1) // CB
NC = 2
NS = 16
NW = NC * NS
PER_W = (B * N) // NW
LW = 128
NCH = PER_W // LW

EPS = 1e-6


def _tc_body(x_ref, w_ref, b_ref, emb_ref, id_ref, out_ref, c_ref, flat_ref):
    out_ref[...] = jnp.full(out_ref.shape, EPS, jnp.float32)

    @pl.when(pl.program_id(1) == 0)
    def _():
        ep = lax.dot_general(
            x_ref[...], w_ref[...], (((1,), (1,)), ((), ())),
            preferred_element_type=jnp.float32,
        ) + b_ref[...]
        logits = jnp.sum(emb_ref[...] * ep[:, None, :], axis=2)
        m = jnp.max(logits, axis=1, keepdims=True)
        e = jnp.exp(logits - m)
        s = e / jnp.sum(e, axis=1, keepdims=True)
        ids = id_ref[...]
        eq = ids[:, :, None] == ids[:, None, :]
        c_ref[...] = jnp.sum(jnp.where(eq, s[:, None, :], 0.0), axis=2) + EPS
        row = pl.program_id(0) * RB + lax.broadcasted_iota(jnp.int32, (RB, N), 0)
        flat_ref[...] = row * V + ids


_tc_fill_score = pl.pallas_call(
    _tc_body,
    grid=(GI, GJ),
    in_specs=[
        pl.BlockSpec((RB, EC), lambda i, j: (i, 0)),
        pl.BlockSpec((EC, EC), lambda i, j: (0, 0)),
        pl.BlockSpec((1, EC), lambda i, j: (0, 0)),
        pl.BlockSpec((RB, N, EC), lambda i, j: (i, 0, 0)),
        pl.BlockSpec((RB, N), lambda i, j: (i, 0)),
    ],
    out_specs=[
        pl.BlockSpec((RB, CB), lambda i, j: (i, j)),
        pl.BlockSpec((RB, N), lambda i, j: (i, 0)),
        pl.BlockSpec((RB, N), lambda i, j: (i, 0)),
    ],
    out_shape=[
        jax.ShapeDtypeStruct((B, V), jnp.float32),
        jax.ShapeDtypeStruct((B, N), jnp.float32),
        jax.ShapeDtypeStruct((B, N), jnp.int32),
    ],
)


@functools.cache
def _get_sc_scatter():
    @functools.partial(
        pl.kernel,
        mesh=plsc.VectorSubcoreMesh(
            core_axis_name="c", subcore_axis_name="s",
            num_cores=NC, num_subcores=NS,
        ),
        out_type=(),
        scratch_types=[
            pltpu.VMEM((NCH, LW), jnp.int32),
            pltpu.VMEM((NCH, LW), jnp.float32),
            pltpu.SemaphoreType.DMA,
        ],
    )
    def _sc_scatter(idx_hbm, val_hbm, big, idx_v, val_v, sem):
        w = lax.axis_index("s") * NC + lax.axis_index("c")
        pltpu.sync_copy(idx_hbm.at[w], idx_v)
        pltpu.sync_copy(val_hbm.at[w], val_v)

        def step(j, carry):
            pltpu.async_copy(val_v.at[j], big.at[idx_v.at[j]], sem).wait()
            return carry

        lax.fori_loop(0, NCH, step, 0)

    return _sc_scatter


def kernel(x, top_city_emb, top_city_id, prob, W, b):
    del prob
    filled, c, flat = _tc_fill_score(
        x, W, b.reshape(1, EC), top_city_emb, top_city_id.astype(jnp.int32)
    )
    ref = jax.new_ref(filled.reshape(B * V))
    _get_sc_scatter()(
        flat.reshape(NW, NCH, LW),
        c.reshape(NW, NCH, LW),
        ref,
    )
    return jax.freeze(ref).reshape(B, V)


# speedup vs baseline: 1.2641x; 1.2641x over previous
import functools

import jax
import jax.numpy as jnp
from jax import lax
from jax.experimental import pallas as pl
from jax.experimental.pallas import tpu as pltpu
from jax.experimental.pallas import tpu_sc as plsc
from jax._src.pallas import mpmd as _mpmd

B = 1024
EC = 32
N = 200
V = 100000
RB = 128
SUB = 8
CB = 12800
GI = B // RB
GJ = (V + CB - 1) // CB
NC = 2
NS = 16
NW = NC * NS
PER_W = (B * N) // NW
LW = 128
NCH = PER_W // LW

EPS = 1e-6


def _tc_body(x_ref, w_ref, b_ref, emb_ref, id_ref, out_ref, c_ref, flat_ref):
    out_ref[...] = jnp.full(out_ref.shape, EPS, jnp.float32)

    @pl.when(pl.program_id(1) == 0)
    def _():
        def sub(k, carry):
            r0 = k * SUB
            xb = x_ref[pl.ds(r0, SUB), :]
            ep = lax.dot_general(
                xb, w_ref[...], (((1,), (1,)), ((), ())),
                preferred_element_type=jnp.float32,
            ) + b_ref[...]
            emb = emb_ref[pl.ds(r0, SUB)]
            logits = jnp.sum(emb * ep[:, None, :], axis=2)
            m = jnp.max(logits, axis=1, keepdims=True)
            e = jnp.exp(logits - m)
            s = e / jnp.sum(e, axis=1, keepdims=True)
            ids = id_ref[pl.ds(r0, SUB), :]
            eq = ids[:, :, None] == ids[:, None, :]
            c = jnp.sum(jnp.where(eq, s[:, None, :], 0.0), axis=2) + EPS
            c_ref[pl.ds(r0, SUB), :] = c
            row = (pl.program_id(0) * RB + r0
                   + lax.broadcasted_iota(jnp.int32, (SUB, N), 0))
            flat_ref[pl.ds(r0, SUB), :] = row * V + ids
            return carry

        lax.fori_loop(0, RB // SUB, sub, 0)


_tc_fill_score = pl.pallas_call(
    _tc_body,
    grid=(GI, GJ),
    in_specs=[
        pl.BlockSpec((RB, EC), lambda i, j: (i, 0)),
        pl.BlockSpec((EC, EC), lambda i, j: (0, 0)),
        pl.BlockSpec((1, EC), lambda i, j: (0, 0)),
        pl.BlockSpec((RB, N, EC), lambda i, j: (i, 0, 0)),
        pl.BlockSpec((RB, N), lambda i, j: (i, 0)),
    ],
    out_specs=[
        pl.BlockSpec((RB, CB), lambda i, j: (i, j)),
        pl.BlockSpec((RB, N), lambda i, j: (i, 0)),
        pl.BlockSpec((RB, N), lambda i, j: (i, 0)),
    ],
    out_shape=[
        jax.ShapeDtypeStruct((B, V), jnp.float32),
        jax.ShapeDtypeStruct((B, N), jnp.float32),
        jax.ShapeDtypeStruct((B, N), jnp.int32),
    ],
)


def _sc_body(idx_hbm, val_hbm, big_in, big_out, idx_v, val_v, sem):
    del big_in
    w = lax.axis_index("s") * NC + lax.axis_index("c")
    pltpu.sync_copy(idx_hbm.at[w], idx_v)
    pltpu.sync_copy(val_hbm.at[w], val_v)

    def fire(j, carry):
        pltpu.async_copy(val_v.at[j], big_out.at[idx_v.at[j]], sem)
        return carry

    lax.fori_loop(0, NCH, fire, 0)

    def drain(j, carry):
        pltpu.make_async_copy(val_v.at[j], big_out.at[idx_v.at[j]], sem).wait()
        return carry

    lax.fori_loop(0, NCH, drain, 0)


@functools.cache
def _get_sc_scatter():
    mesh = plsc.VectorSubcoreMesh(
        core_axis_name="c", subcore_axis_name="s",
        num_cores=NC, num_subcores=NS,
    )
    return _mpmd._mpmd_map(
        [(mesh, _sc_body)],
        jax.ShapeDtypeStruct((B * V,), jnp.float32),
        input_output_aliases={2: 0},
        scratch_types=[
            pltpu.VMEM((NCH, LW), jnp.int32),
            pltpu.VMEM((NCH, LW), jnp.float32),
            pltpu.SemaphoreType.DMA,
        ],
    )


def kernel(x, top_city_emb, top_city_id, prob, W, b):
    del prob
    filled, c, flat = _tc_fill_score(
        x, W, b.reshape(1, EC), top_city_emb, top_city_id.astype(jnp.int32)
    )
    out = _get_sc_scatter()(
        flat.reshape(NW, NCH, LW),
        c.reshape(NW, NCH, LW),
        filled.reshape(B * V),
    )
    return out.reshape(B, V)


# speedup vs baseline: 2.9083x; 2.3008x over previous
import functools

import jax
import jax.numpy as jnp
from jax import lax
from jax.experimental import pallas as pl
from jax.experimental.pallas import tpu as pltpu
from jax.experimental.pallas import tpu_sc as plsc
from jax._src.pallas import mpmd as _mpmd

B = 1024
EC = 32
N = 200
NP = 256
V = 100000
VS = 99968
RB = 8
GI = B // RB
NC = 2
NS = 16
NW = NC * NS
RPW = B // NW
L = 16
TR = 8
RGPW = RPW // TR
CW = 12800
CHUNKS = [(c0, min(CW, VS - c0)) for c0 in range(0, VS, CW)]

EPS = 1e-6


def _tc_body(x_ref, w_ref, b_ref, emb_ref, id_ref, c_ref, idp_ref, strip_ref):
    ep = lax.dot_general(
        x_ref[...], w_ref[...], (((1,), (1,)), ((), ())),
        preferred_element_type=jnp.float32,
    ) + b_ref[...]
    logits = jnp.sum(emb_ref[...] * ep[:, None, :], axis=2)
    m = jnp.max(logits, axis=1, keepdims=True)
    e = jnp.exp(logits - m)
    s = e / jnp.sum(e, axis=1, keepdims=True)
    ids = id_ref[...]
    idp = jnp.concatenate(
        [ids, jnp.broadcast_to(ids[:, :1], (RB, NP - N))], axis=1)
    sp = jnp.concatenate([s, jnp.zeros((RB, NP - N), jnp.float32)], axis=1)
    eq = idp[:, :, None] == idp[:, None, :]
    c_ref[...] = jnp.sum(jnp.where(eq, sp[:, None, :], 0.0), axis=2) + EPS
    idp_ref[...] = idp
    col = VS + lax.broadcasted_iota(jnp.int32, (RB, NP, 128), 2)
    hit = idp[:, :, None] == col
    strip_ref[...] = jnp.sum(jnp.where(hit, sp[:, :, None], 0.0), axis=1) + EPS


_tc_score = pl.pallas_call(
    _tc_body,
    grid=(GI,),
    in_specs=[
        pl.BlockSpec((RB, EC), lambda i: (i, 0)),
        pl.BlockSpec((EC, EC), lambda i: (0, 0)),
        pl.BlockSpec((1, EC), lambda i: (0, 0)),
        pl.BlockSpec((RB, N, EC), lambda i: (i, 0, 0)),
        pl.BlockSpec((RB, N), lambda i: (i, 0)),
    ],
    out_specs=[
        pl.BlockSpec((RB, NP), lambda i: (i, 0)),
        pl.BlockSpec((RB, NP), lambda i: (i, 0)),
        pl.BlockSpec((RB, 128), lambda i: (i, VS // 128)),
    ],
    out_shape=[
        jax.ShapeDtypeStruct((B, NP), jnp.float32),
        jax.ShapeDtypeStruct((B, NP), jnp.int32),
        jax.ShapeDtypeStruct((B, V), jnp.float32),
    ],
)


def _sc_body(idx_hbm, val_hbm, big_in, out_hbm, idx_v, val_v, buf, sem_in, sem):
    del big_in
    w = lax.axis_index("s") * NC + lax.axis_index("c")
    pltpu.async_copy(idx_hbm.at[w], idx_v, sem_in).wait()
    pltpu.async_copy(val_hbm.at[w], val_v, sem_in).wait()

    eps16 = jnp.full((L,), EPS, jnp.float32)

    def fill(k, carry):
        for r in range(TR):
            buf[r, pl.ds(k * L, L)] = eps16
        return carry

    lax.fori_loop(0, CW // L, fill, 0)

    def sweep(base, c0, csz, value_of):
        def srow(r, carry):
            rr = jnp.full((L,), r, jnp.int32)
            for k in range(NP // L):
                idx = idx_v[base + r, pl.ds(k * L, L)]
                val = value_of(val_v[base + r, pl.ds(k * L, L)])
                msk = (idx >= c0) & (idx < c0 + csz)
                loc = jnp.where(msk, idx - c0, 0)
                plsc.store_scatter(buf, [rr, loc], val, mask=msk)
            return carry

        lax.fori_loop(0, TR, srow, 0)

    def group(g, carry):
        base = g * TR
        grow = pl.multiple_of(w * RPW + base, TR)
        for c0, csz in CHUNKS:
            sweep(base, c0, csz, lambda v: v)
            pltpu.async_copy(
                buf.at[:, pl.ds(0, csz)],
                out_hbm.at[pl.ds(grow, TR), pl.ds(c0, csz)],
                sem,
            ).wait()
            sweep(base, c0, csz, lambda v: eps16)
        return carry

    lax.fori_loop(0, RGPW, group, 0)


@functools.cache
def _get_sc_fill_scatter():
    mesh = plsc.VectorSubcoreMesh(
        core_axis_name="c", subcore_axis_name="s",
        num_cores=NC, num_subcores=NS,
    )
    return _mpmd._mpmd_map(
        [(mesh, _sc_body)],
        jax.ShapeDtypeStruct((B, V), jnp.float32),
        input_output_aliases={2: 0},
        compiler_params=pltpu.CompilerParams(needs_layout_passes=False),
        scratch_types=[
            pltpu.VMEM((RPW, NP), jnp.int32),
            pltpu.VMEM((RPW, NP), jnp.float32),
            pltpu.VMEM((TR, CW), jnp.float32),
            pltpu.SemaphoreType.DMA,
            pltpu.SemaphoreType.DMA,
        ],
    )


def kernel(x, top_city_emb, top_city_id, prob, W, b):
    del prob
    c, idp, big = _tc_score(
        x, W, b.reshape(1, EC), top_city_emb, top_city_id.astype(jnp.int32)
    )
    out = _get_sc_fill_scatter()(
        idp.reshape(NW, RPW, NP),
        c.reshape(NW, RPW, NP),
        big,
    )
    return out


# speedup vs baseline: 3.1913x; 1.0973x over previous
import functools

import jax
import jax.numpy as jnp
from jax import lax
from jax.experimental import pallas as pl
from jax.experimental.pallas import tpu as pltpu
from jax.experimental.pallas import tpu_sc as plsc
from jax._src.pallas import mpmd as _mpmd

B = 1024
EC = 32
N = 200
NP = 256
V = 100000
VS = 99968
RB = 16
GI = B // RB
NC = 2
NS = 16
NW = NC * NS
RPW = B // NW
L = 16
TR = 8
RGPW = RPW // TR
CW = 12800
CHUNKS = [(c0, min(CW, VS - c0)) for c0 in range(0, VS, CW)]

EPS = 1e-6


def _tc_body(x_ref, w_ref, b_ref, emb_ref, id_ref, c_ref, idp_ref, strip_ref):
    ep = lax.dot_general(
        x_ref[...], w_ref[...], (((1,), (1,)), ((), ())),
        preferred_element_type=jnp.float32,
    ) + b_ref[...]
    logits = jnp.sum(emb_ref[...] * ep[:, None, :], axis=2)
    m = jnp.max(logits, axis=1, keepdims=True)
    e = jnp.exp(logits - m)
    s = e / jnp.sum(e, axis=1, keepdims=True)
    ids = id_ref[...]
    idp = jnp.concatenate(
        [ids, jnp.broadcast_to(ids[:, :1], (RB, NP - N))], axis=1)
    sp = jnp.concatenate([s, jnp.zeros((RB, NP - N), jnp.float32)], axis=1)
    c_ref[...] = sp
    idp_ref[...] = idp
    col = VS + lax.broadcasted_iota(jnp.int32, (RB, NP, 128), 2)
    hit = idp[:, :, None] == col
    strip_ref[...] = jnp.sum(jnp.where(hit, sp[:, :, None], 0.0), axis=1) + EPS


_tc_score = pl.pallas_call(
    _tc_body,
    grid=(GI,),
    in_specs=[
        pl.BlockSpec((RB, EC), lambda i: (i, 0)),
        pl.BlockSpec((EC, EC), lambda i: (0, 0)),
        pl.BlockSpec((1, EC), lambda i: (0, 0)),
        pl.BlockSpec((RB, N, EC), lambda i: (i, 0, 0)),
        pl.BlockSpec((RB, N), lambda i: (i, 0)),
    ],
    out_specs=[
        pl.BlockSpec((RB, NP), lambda i: (i, 0)),
        pl.BlockSpec((RB, NP), lambda i: (i, 0)),
        pl.BlockSpec((RB, 128), lambda i: (i, VS // 128)),
    ],
    out_shape=[
        jax.ShapeDtypeStruct((B, NP), jnp.float32),
        jax.ShapeDtypeStruct((B, NP), jnp.int32),
        jax.ShapeDtypeStruct((B, V), jnp.float32),
    ],
)


def _sc_body(idx_hbm, val_hbm, big_in, out_hbm, idx_v, val_v, buf, sem_in, sem):
    del big_in
    w = lax.axis_index("s") * NC + lax.axis_index("c")
    pltpu.async_copy(idx_hbm.at[w], idx_v, sem_in).wait()
    pltpu.async_copy(val_hbm.at[w], val_v, sem_in).wait()

    eps16 = jnp.full((L,), EPS, jnp.float32)

    def fill(k, carry):
        for r in range(TR):
            buf[r, pl.ds(k * L, L)] = eps16
        return carry

    lax.fori_loop(0, CW // L, fill, 0)

    def sweep(base, c0, csz, restore):
        def srow(r, carry):
            rr = jnp.full((L,), r, jnp.int32)
            for k in range(NP // L):
                idx = idx_v[base + r, pl.ds(k * L, L)]
                msk = (idx >= c0) & (idx < c0 + csz)
                loc = jnp.where(msk, idx - c0, 0)
                if restore:
                    plsc.store_scatter(buf, [rr, loc], eps16, mask=msk)
                else:
                    val = val_v[base + r, pl.ds(k * L, L)]
                    plsc.addupdate_scatter(buf, [rr, loc], val, mask=msk)
            return carry

        lax.fori_loop(0, TR, srow, 0)

    def group(g, carry):
        base = g * TR
        grow = pl.multiple_of(w * RPW + base, TR)
        for c0, csz in CHUNKS:
            sweep(base, c0, csz, restore=False)
            pltpu.async_copy(
                buf.at[:, pl.ds(0, csz)],
                out_hbm.at[pl.ds(grow, TR), pl.ds(c0, csz)],
                sem,
            ).wait()
            sweep(base, c0, csz, restore=True)
        return carry

    lax.fori_loop(0, RGPW, group, 0)


@functools.cache
def _get_sc_fill_scatter():
    mesh = plsc.VectorSubcoreMesh(
        core_axis_name="c", subcore_axis_name="s",
        num_cores=NC, num_subcores=NS,
    )
    return _mpmd._mpmd_map(
        [(mesh, _sc_body)],
        jax.ShapeDtypeStruct((B, V), jnp.float32),
        input_output_aliases={2: 0},
        compiler_params=pltpu.CompilerParams(needs_layout_passes=False),
        scratch_types=[
            pltpu.VMEM((RPW, NP), jnp.int32),
            pltpu.VMEM((RPW, NP), jnp.float32),
            pltpu.VMEM((TR, CW), jnp.float32),
            pltpu.SemaphoreType.DMA,
            pltpu.SemaphoreType.DMA,
        ],
    )


def kernel(x, top_city_emb, top_city_id, prob, W, b):
    del prob
    c, idp, big = _tc_score(
        x, W, b.reshape(1, EC), top_city_emb, top_city_id.astype(jnp.int32)
    )
    out = _get_sc_fill_scatter()(
        idp.reshape(NW, RPW, NP),
        c.reshape(NW, RPW, NP),
        big,
    )
    return out


# speedup vs baseline: 3.8416x; 1.2038x over previous
import functools

import jax
import jax.numpy as jnp
from jax import lax
from jax.experimental import pallas as pl
from jax.experimental.pallas import tpu as pltpu
from jax.experimental.pallas import tpu_sc as plsc

B = 1024
EC = 32
N = 200
NP = 256
V = 100000
VP = 100096
RB = 16
GI = B // RB
NC = 2
NS = 16
NW = NC * NS
RPW = B // NW
L = 16
TR = 8
RGPW = RPW // TR
CW = 12800
CHUNKS = [(c0, min(CW, VP - c0)) for c0 in range(0, VP, CW)]

EPS = 1e-6


def _tc_body(x_ref, w_ref, b_ref, emb_ref, id_ref, c_ref, idp_ref):
    ep = lax.dot_general(
        x_ref[...], w_ref[...], (((1,), (1,)), ((), ())),
        preferred_element_type=jnp.float32,
    ) + b_ref[...]
    logits = jnp.sum(emb_ref[...] * ep[:, None, :], axis=2)
    m = jnp.max(logits, axis=1, keepdims=True)
    e = jnp.exp(logits - m)
    s = e / jnp.sum(e, axis=1, keepdims=True)
    ids = id_ref[...]
    idp = jnp.concatenate(
        [ids, jnp.broadcast_to(ids[:, :1], (RB, NP - N))], axis=1)
    sp = jnp.concatenate([s, jnp.zeros((RB, NP - N), jnp.float32)], axis=1)
    c_ref[...] = sp
    idp_ref[...] = idp


_tc_score = pl.pallas_call(
    _tc_body,
    grid=(GI,),
    in_specs=[
        pl.BlockSpec((RB, EC), lambda i: (i, 0)),
        pl.BlockSpec((EC, EC), lambda i: (0, 0)),
        pl.BlockSpec((1, EC), lambda i: (0, 0)),
        pl.BlockSpec((RB, N, EC), lambda i: (i, 0, 0)),
        pl.BlockSpec((RB, N), lambda i: (i, 0)),
    ],
    out_specs=[
        pl.BlockSpec((RB, NP), lambda i: (i, 0)),
        pl.BlockSpec((RB, NP), lambda i: (i, 0)),
    ],
    out_shape=[
        jax.ShapeDtypeStruct((B, NP), jnp.float32),
        jax.ShapeDtypeStruct((B, NP), jnp.int32),
    ],
)


def _sc_body(idx_hbm, val_hbm, out_hbm, idx_v, val_v, buf, sem_in, sem):
    w = lax.axis_index("s") * NC + lax.axis_index("c")
    pltpu.async_copy(idx_hbm.at[w], idx_v, sem_in).wait()
    pltpu.async_copy(val_hbm.at[w], val_v, sem_in).wait()

    eps16 = jnp.full((L,), EPS, jnp.float32)

    def fill(k, carry):
        for r in range(TR):
            buf[r, pl.ds(k * L, L)] = eps16
        return carry

    lax.fori_loop(0, CW // L, fill, 0)

    def sweep(base, c0, csz, restore):
        def srow(r, carry):
            rr = jnp.full((L,), r, jnp.int32)
            for k in range(NP // L):
                idx = idx_v[base + r, pl.ds(k * L, L)]
                msk = (idx >= c0) & (idx < c0 + csz)
                loc = jnp.where(msk, idx - c0, 0)
                if restore:
                    plsc.store_scatter(buf, [rr, loc], eps16, mask=msk)
                else:
                    val = val_v[base + r, pl.ds(k * L, L)]
                    plsc.addupdate_scatter(buf, [rr, loc], val, mask=msk)
            return carry

        lax.fori_loop(0, TR, srow, 0)

    def group(g, carry):
        base = g * TR
        grow = pl.multiple_of(w * RPW + base, TR)
        for c0, csz in CHUNKS:
            sweep(base, c0, csz, restore=False)
            pltpu.async_copy(
                buf.at[:, pl.ds(0, csz)],
                out_hbm.at[pl.ds(grow, TR), pl.ds(c0, csz)],
                sem,
            ).wait()
            sweep(base, c0, csz, restore=True)
        return carry

    lax.fori_loop(0, RGPW, group, 0)


@functools.cache
def _get_sc_fill_scatter():
    mesh = plsc.VectorSubcoreMesh(
        core_axis_name="c", subcore_axis_name="s",
        num_cores=NC, num_subcores=NS,
    )
    return pl.kernel(
        _sc_body,
        mesh=mesh,
        out_type=jax.ShapeDtypeStruct((B, VP), jnp.float32),
        compiler_params=pltpu.CompilerParams(needs_layout_passes=False),
        scratch_types=[
            pltpu.VMEM((RPW, NP), jnp.int32),
            pltpu.VMEM((RPW, NP), jnp.float32),
            pltpu.VMEM((TR, CW), jnp.float32),
            pltpu.SemaphoreType.DMA,
            pltpu.SemaphoreType.DMA,
        ],
    )


def kernel(x, top_city_emb, top_city_id, prob, W, b):
    del prob
    c, idp = _tc_score(
        x, W, b.reshape(1, EC), top_city_emb, top_city_id.astype(jnp.int32)
    )
    out = _get_sc_fill_scatter()(
        idp.reshape(NW, RPW, NP),
        c.reshape(NW, RPW, NP),
    )
    return out[:, :V]
